# baseline (speedup 1.0000x reference)
import functools

import jax
import jax.numpy as jnp
from jax import lax
from jax.experimental import pallas as pl
from jax.experimental.pallas import tpu as pltpu
from jax.experimental.pallas import tpu_sc as plsc

N = 10000
E = 160000
D_IN = 256
D_OUT = 256
H = 128
NT = 16
EPT = E // NT
CH = 16
NCH = EPT // CH
NPAIR = (NCH - 1) // 2
NRC = N // 16
RPT = NRC // NT


def _rsqrt16(d):
    i = lax.bitcast_convert_type(d, jnp.int32)
    i = jnp.int32(0x5F3759DF) - lax.shift_right_logical(i, 1)
    y = lax.bitcast_convert_type(i, jnp.float32)
    for _ in range(3):
        y = y * (1.5 - 0.5 * d * y * y)
    return y


def _mm_body(x_ref, w_ref, o_ref):
    o_ref[0] = jnp.dot(x_ref[...], w_ref[...], preferred_element_type=jnp.float32)


def _xlin_split(x, W):
    BN = 400
    out = pl.pallas_call(
        _mm_body,
        grid=(2, N // BN),
        in_specs=[
            pl.BlockSpec((BN, D_IN), lambda h, i: (i, 0)),
            pl.BlockSpec((D_IN, H), lambda h, i: (0, h)),
        ],
        out_specs=pl.BlockSpec((1, BN, H), lambda h, i: (h, i, 0)),
        out_shape=jax.ShapeDtypeStruct((2, N, H), jnp.float32),
    )(x, W)
    return out.reshape(2 * N, H)


_mesh = plsc.VectorSubcoreMesh(core_axis_name="c", subcore_axis_name="s")


@functools.partial(
    pl.kernel,
    out_type=jax.ShapeDtypeStruct((N, D_OUT), jnp.float32),
    mesh=_mesh,
    compiler_params=pltpu.CompilerParams(
        needs_layout_passes=False,
        use_tc_tiling_on_sc=False,
    ),
    scratch_types=[
        pltpu.VMEM_SHARED((N, H), jnp.float32),
        pltpu.VMEM_SHARED((N,), jnp.float32),
        pltpu.VMEM((N,), jnp.float32),
        pltpu.VMEM((NCH, CH), jnp.int32),
        pltpu.VMEM((NCH, CH), jnp.int32),
        pltpu.VMEM((NCH, CH), jnp.float32),
        pltpu.VMEM((CH, H), jnp.float32),
        pltpu.VMEM((CH, H), jnp.float32),
        pltpu.VMEM((16,), jnp.float32),
        pltpu.VMEM((H,), jnp.float32),
        pltpu.SemaphoreType.DMA,
        pltpu.SemaphoreType.DMA,
        pltpu.SemaphoreType.DMA,
        pltpu.SemaphoreType.DMA,
    ],
)
def _gcn_sc(xlin, row3, col3, ew3, zeros_n, bvec, out,
            acc_spm, dg_spm, dis_v, row_a, col_a, ew_a,
            gbuf_a, gbuf_b, dtmp, b_vm, sem_ga, sem_gb, sem_sa, sem_sb):
    c_idx = lax.axis_index("c")
    s_idx = lax.axis_index("s")
    half_base = c_idx * N

    pltpu.sync_copy(bvec.at[pl.ds(c_idx * H, H)], b_vm)
    pltpu.sync_copy(row3.at[s_idx], row_a)
    pltpu.sync_copy(col3.at[s_idx], col_a)
    pltpu.sync_copy(ew3.at[s_idx], ew_a)

    @pl.when(s_idx == 0)
    def _():
        pltpu.sync_copy(zeros_n, dg_spm)

    plsc.subcore_barrier()

    def deg_group(g, carry):
        base = g * 8
        for i in range(8):
            pltpu.async_copy(ew_a.at[base + i], dg_spm.at[col_a.at[base + i]],
                             sem_ga, add=True)
        for i in range(8):
            pltpu.make_async_copy(ew_a.at[base + i],
                                  dg_spm.at[col_a.at[base + i]], sem_ga).wait()
        return carry

    lax.fori_loop(0, NCH // 8, deg_group, 0)
    pltpu.sync_copy(ew_a.at[NCH - 1], dg_spm.at[col_a.at[NCH - 1]], add=True)
    plsc.subcore_barrier()

    def dis_chunk(k, carry):
        st = pl.multiple_of(k * 16, 16)
        pltpu.sync_copy(dg_spm.at[pl.ds(st, 16)], dtmp)
        dtmp[...] = _rsqrt16(dtmp[...] + 1.0)
        pltpu.sync_copy(dtmp, dg_spm.at[pl.ds(st, 16)])
        return carry

    lax.fori_loop(s_idx * RPT, (s_idx + 1) * RPT, dis_chunk, 0)

    @pl.when(s_idx == NT - 1)
    def _():
        dis_chunk(NRC - 1, 0)

    plsc.subcore_barrier()

    pltpu.sync_copy(dg_spm, dis_v)

    def prep_chunk(u, carry):
        r16 = row_a[u]
        c16 = col_a[u]
        w16 = ew_a[u]
        dr = plsc.load_gather(dis_v, [r16])
        dc = plsc.load_gather(dis_v, [c16])
        ew_a[u] = dr * w16 * dc
        row_a[u] = r16 + half_base
        return carry

    lax.fori_loop(0, NCH, prep_chunk, 0)

    def b2_issue(k, buf, sem):
        st = pl.multiple_of(k * 16, 16)
        pltpu.async_copy(xlin.at[pl.ds(half_base + st, 16)], buf, sem)

    def b2_wait(k, buf, sem):
        st = pl.multiple_of(k * 16, 16)
        pltpu.make_async_copy(xlin.at[pl.ds(half_base + st, 16)], buf, sem).wait()

    def b2_process(k, buf):
        st = pl.multiple_of(k * 16, 16)
        dv = dis_v[pl.ds(st, 16)]
        for e in range(16):
            dse = dv[e]
            s2 = dse * dse
            for q in range(H // 16):
                sl = pl.ds(q * 16, 16)
                buf[e, sl] = buf[e, sl] * s2
        pltpu.sync_copy(buf, acc_spm.at[pl.ds(st, 16)])

    k0 = s_idx * RPT
    b2_issue(k0, gbuf_a, sem_ga)

    def b2_pair(q, carry):
        ka = k0 + q * 2
        b2_wait(ka, gbuf_a, sem_ga)
        b2_issue(ka + 1, gbuf_b, sem_gb)
        b2_process(ka, gbuf_a)
        b2_wait(ka + 1, gbuf_b, sem_gb)
        b2_issue(ka + 2, gbuf_a, sem_ga)
        b2_process(ka + 1, gbuf_b)
        return carry

    lax.fori_loop(0, RPT // 2, b2_pair, 0)
    b2_wait(k0 + RPT - 1, gbuf_a, sem_ga)
    b2_process(k0 + RPT - 1, gbuf_a)

    @pl.when(s_idx == NT - 1)
    def _():
        st = pl.multiple_of((NRC - 1) * 16, 16)
        pltpu.sync_copy(xlin.at[pl.ds(half_base + st, 16)], gbuf_a)
        b2_process(NRC - 1, gbuf_a)

    plsc.subcore_barrier()

    def issue(u, buf, sem):
        pltpu.async_copy(xlin.at[row_a[u]], buf, sem)

    def wait_g(u, buf, sem):
        pltpu.make_async_copy(xlin.at[row_a[u]], buf, sem).wait()

    def scale(u, buf):
        norm16 = ew_a[u]
        for e in range(CH):
            ne = norm16[e]
            for q in range(H // 16):
                sl = pl.ds(q * 16, 16)
                buf[e, sl] = buf[e, sl] * ne

    def scat(u, buf, sem):
        pltpu.async_copy(buf, acc_spm.at[col_a.at[u]], sem, add=True)

    def wait_s(buf, sem):
        pltpu.make_async_copy(buf, acc_spm.at[col_a.at[0]], sem).wait()

    issue(0, gbuf_a, sem_ga)

    def pair_body(p, carry):
        u0 = p * 2
        wait_g(u0, gbuf_a, sem_ga)
        scale(u0, gbuf_a)
        scat(u0, gbuf_a, sem_sa)

        @pl.when(p > 0)
        def _():
            wait_s(gbuf_b, sem_sb)

        issue(u0 + 1, gbuf_b, sem_gb)
        wait_g(u0 + 1, gbuf_b, sem_gb)
        scale(u0 + 1, gbuf_b)
        scat(u0 + 1, gbuf_b, sem_sb)
        wait_s(gbuf_a, sem_sa)
        issue(u0 + 2, gbuf_a, sem_ga)
        return carry

    lax.fori_loop(0, NPAIR, pair_body, 0)
    wait_g(NCH - 1, gbuf_a, sem_ga)
    scale(NCH - 1, gbuf_a)
    scat(NCH - 1, gbuf_a, sem_sa)
    wait_s(gbuf_b, sem_sb)
    wait_s(gbuf_a, sem_sa)
    plsc.subcore_barrier()

    def d_read(k, buf):
        st = pl.multiple_of(k * 16, 16)
        pltpu.sync_copy(acc_spm.at[pl.ds(st, 16)], buf)
        for e in range(16):
            for q in range(H // 16):
                sl = pl.ds(q * 16, 16)
                buf[e, sl] = jnp.maximum(buf[e, sl] + b_vm[sl], 0.0)

    def d_write(k, buf, sem):
        st = pl.multiple_of(k * 16, 16)
        pltpu.async_copy(buf, out.at[pl.ds(st, 16), pl.ds(c_idx * H, H)], sem)

    def d_wait(k, buf, sem):
        st = pl.multiple_of(k * 16, 16)
        pltpu.make_async_copy(buf, out.at[pl.ds(st, 16), pl.ds(c_idx * H, H)],
                              sem).wait()

    def d_pair(q, carry):
        ka = k0 + q * 2

        @pl.when(q > 0)
        def _():
            d_wait(ka - 2, gbuf_a, sem_ga)

        d_read(ka, gbuf_a)
        d_write(ka, gbuf_a, sem_ga)

        @pl.when(q > 0)
        def _():
            d_wait(ka - 1, gbuf_b, sem_gb)

        d_read(ka + 1, gbuf_b)
        d_write(ka + 1, gbuf_b, sem_gb)
        return carry

    lax.fori_loop(0, RPT // 2, d_pair, 0)
    d_wait(k0 + RPT - 3, gbuf_a, sem_ga)
    d_read(k0 + RPT - 1, gbuf_a)
    d_write(k0 + RPT - 1, gbuf_a, sem_ga)

    @pl.when(s_idx == NT - 1)
    def _():
        d_wait(k0 + RPT - 2, gbuf_b, sem_gb)
        d_read(NRC - 1, gbuf_b)
        d_write(NRC - 1, gbuf_b, sem_gb)

    d_wait(k0 + RPT - 1, gbuf_a, sem_ga)

    @pl.when(s_idx == NT - 1)
    def _():
        d_wait(NRC - 1, gbuf_b, sem_gb)

    @pl.when(s_idx != NT - 1)
    def _():
        d_wait(k0 + RPT - 2, gbuf_b, sem_gb)


def kernel(x, edge_index, edge_weight, W, b):
    x = x.astype(jnp.float32)
    W = W.astype(jnp.float32)
    ew = edge_weight.astype(jnp.float32)
    b = b.astype(jnp.float32)
    row = edge_index[0].astype(jnp.int32)
    col = edge_index[1].astype(jnp.int32)

    xlin = _xlin_split(x, W)
    row3 = row.reshape(NT, NCH, CH)
    col3 = col.reshape(NT, NCH, CH)
    ew3 = ew.reshape(NT, NCH, CH)
    zeros_n = jnp.zeros((N,), jnp.float32)

    return _gcn_sc(xlin, row3, col3, ew3, zeros_n, b)


# speedup vs baseline: 1.8531x; 1.8531x over previous
import functools

import jax
import jax.numpy as jnp
from jax import lax
from jax.experimental import pallas as pl
from jax.experimental.pallas import tpu as pltpu
from jax.experimental.pallas import tpu_sc as plsc

N = 10000
E = 160000
D_IN = 256
D_OUT = 256
H = 128
NT = 16
EPT = E // NT
CH = 16
NP = 5
NPC = EPT // NP // CH
NRC = N // 16
RPT = NRC // NT


def _rsqrt16(d):
    i = lax.bitcast_convert_type(d, jnp.int32)
    i = jnp.int32(0x5F3759DF) - lax.shift_right_logical(i, 1)
    y = lax.bitcast_convert_type(i, jnp.float32)
    for _ in range(3):
        y = y * (1.5 - 0.5 * d * y * y)
    return y


def _mm_body(x_ref, w_ref, o_ref):
    o_ref[0] = jnp.dot(x_ref[...], w_ref[...], preferred_element_type=jnp.float32)


def _xlin_split(x, W):
    BN = 400
    out = pl.pallas_call(
        _mm_body,
        grid=(2, N // BN),
        in_specs=[
            pl.BlockSpec((BN, D_IN), lambda h, i: (i, 0)),
            pl.BlockSpec((D_IN, H), lambda h, i: (0, h)),
        ],
        out_specs=pl.BlockSpec((1, BN, H), lambda h, i: (h, i, 0)),
        out_shape=jax.ShapeDtypeStruct((2, N, H), jnp.float32),
    )(x, W)
    return out.reshape(2 * N, H)


_mesh = plsc.VectorSubcoreMesh(core_axis_name="c", subcore_axis_name="s")


@functools.partial(
    pl.kernel,
    out_type=jax.ShapeDtypeStruct((N, D_OUT), jnp.float32),
    mesh=_mesh,
    compiler_params=pltpu.CompilerParams(
        needs_layout_passes=False,
        use_tc_tiling_on_sc=False,
    ),
    scratch_types=[
        pltpu.VMEM_SHARED((N, H), jnp.float32),
        pltpu.VMEM_SHARED((N,), jnp.float32),
        pltpu.VMEM((N,), jnp.float32),
        pltpu.VMEM((NPC, CH), jnp.int32),
        pltpu.VMEM((NPC, CH), jnp.int32),
        pltpu.VMEM((NPC, CH), jnp.float32),
        pltpu.VMEM((CH, H), jnp.float32),
        pltpu.VMEM((CH, H), jnp.float32),
        pltpu.VMEM((CH, H), jnp.float32),
        pltpu.VMEM((CH, H), jnp.float32),
        pltpu.VMEM((16,), jnp.float32),
        pltpu.VMEM((H,), jnp.float32),
        pltpu.SemaphoreType.DMA,
        pltpu.SemaphoreType.DMA,
        pltpu.SemaphoreType.DMA,
        pltpu.SemaphoreType.DMA,
    ],
)
def _gcn_sc(xlin, row4, col4, ew4, zeros_n, bvec, out,
            acc_spm, dg_spm, dis_v, row_a, col_a, ew_a,
            gbuf_a, gbuf_b, gbuf_c, sbuf, dtmp, b_vm,
            sem_ga, sem_gb, sem_gc, sem_sa):
    c_idx = lax.axis_index("c")
    s_idx = lax.axis_index("s")
    half_base = c_idx * N

    pltpu.sync_copy(bvec.at[pl.ds(c_idx * H, H)], b_vm)

    @pl.when(s_idx == 0)
    def _():
        pltpu.sync_copy(zeros_n, dg_spm)

    plsc.subcore_barrier()

    def deg_part(hh, carry):
        pltpu.sync_copy(col4.at[s_idx, hh], col_a)
        pltpu.sync_copy(ew4.at[s_idx, hh], ew_a)

        def deg_group(g, carry2):
            base = g * 8
            for i in range(8):
                pltpu.async_copy(ew_a.at[base + i], dg_spm.at[col_a.at[base + i]],
                                 sem_ga, add=True)
            for i in range(8):
                pltpu.make_async_copy(ew_a.at[base + i],
                                      dg_spm.at[col_a.at[base + i]], sem_ga).wait()
            return carry2

        lax.fori_loop(0, NPC // 8, deg_group, 0)
        for u in range(NPC - NPC // 8 * 8):
            pltpu.sync_copy(ew_a.at[NPC // 8 * 8 + u],
                            dg_spm.at[col_a.at[NPC // 8 * 8 + u]], add=True)
        return carry

    lax.fori_loop(0, NP, deg_part, 0)
    plsc.subcore_barrier()

    def dis_chunk(k, carry):
        st = pl.multiple_of(k * 16, 16)
        pltpu.sync_copy(dg_spm.at[pl.ds(st, 16)], dtmp)
        dtmp[...] = _rsqrt16(dtmp[...] + 1.0)
        pltpu.sync_copy(dtmp, dg_spm.at[pl.ds(st, 16)])
        return carry

    lax.fori_loop(s_idx * RPT, (s_idx + 1) * RPT, dis_chunk, 0)

    @pl.when(s_idx == NT - 1)
    def _():
        dis_chunk(NRC - 1, 0)

    plsc.subcore_barrier()

    pltpu.sync_copy(dg_spm, dis_v)

    def b2_issue(k, buf, sem):
        st = pl.multiple_of(k * 16, 16)
        pltpu.async_copy(xlin.at[pl.ds(half_base + st, 16)], buf, sem)

    def b2_wait(k, buf, sem):
        st = pl.multiple_of(k * 16, 16)
        pltpu.make_async_copy(xlin.at[pl.ds(half_base + st, 16)], buf, sem).wait()

    def b2_process(k, buf):
        st = pl.multiple_of(k * 16, 16)
        dv = dis_v[pl.ds(st, 16)]
        for e in range(16):
            dse = dv[e]
            s2 = dse * dse
            for q in range(H // 16):
                sl = pl.ds(q * 16, 16)
                buf[e, sl] = buf[e, sl] * s2
        pltpu.sync_copy(buf, acc_spm.at[pl.ds(st, 16)])

    k0 = s_idx * RPT
    b2_issue(k0, gbuf_a, sem_ga)

    def b2_pair(q, carry):
        ka = k0 + q * 2
        b2_wait(ka, gbuf_a, sem_ga)
        b2_issue(ka + 1, gbuf_b, sem_gb)
        b2_process(ka, gbuf_a)
        b2_wait(ka + 1, gbuf_b, sem_gb)
        b2_issue(ka + 2, gbuf_a, sem_ga)
        b2_process(ka + 1, gbuf_b)
        return carry

    lax.fori_loop(0, RPT // 2, b2_pair, 0)
    b2_wait(k0 + RPT - 1, gbuf_a, sem_ga)
    b2_process(k0 + RPT - 1, gbuf_a)

    @pl.when(s_idx == NT - 1)
    def _():
        st = pl.multiple_of((NRC - 1) * 16, 16)
        pltpu.sync_copy(xlin.at[pl.ds(half_base + st, 16)], gbuf_a)
        b2_process(NRC - 1, gbuf_a)

    plsc.subcore_barrier()

    def issue(u, buf, sem):
        pltpu.async_copy(xlin.at[row_a[u]], buf, sem)

    def wait_g(u, buf, sem):
        pltpu.make_async_copy(xlin.at[row_a[u]], buf, sem).wait()

    def scat(u, buf, sem):
        pltpu.async_copy(buf, acc_spm.at[col_a.at[u]], sem, add=True)

    def wait_s(buf, sem):
        pltpu.make_async_copy(buf, acc_spm.at[col_a.at[0]], sem).wait()

    gbufs = (gbuf_a, gbuf_b, gbuf_c)
    gsems = (sem_ga, sem_gb, sem_gc)

    def slot(u, gi):
        gb, gs = gbufs[gi], gsems[gi]
        wait_g(u, gb, gs)

        @pl.when(u >= 1)
        def _():
            wait_s(sbuf, sem_sa)

        norm16 = ew_a[u]
        for e in range(CH):
            ne = norm16[e]
            for q in range(H // 16):
                sl = pl.ds(q * 16, 16)
                sbuf[e, sl] = gb[e, sl] * ne

        @pl.when(u + 3 < NPC)
        def _():
            issue(u + 3, gb, gs)

        scat(u, sbuf, sem_sa)

    def c_part(hh, carry):
        pltpu.sync_copy(row4.at[s_idx, hh], row_a)
        pltpu.sync_copy(col4.at[s_idx, hh], col_a)
        pltpu.sync_copy(ew4.at[s_idx, hh], ew_a)

        def prep_chunk(u, carry2):
            r16 = row_a[u]
            c16 = col_a[u]
            w16 = ew_a[u]
            dr = plsc.load_gather(dis_v, [r16])
            dc = plsc.load_gather(dis_v, [c16])
            ew_a[u] = dr * w16 * dc
            row_a[u] = r16 + half_base
            return carry2

        lax.fori_loop(0, NPC, prep_chunk, 0)

        issue(0, gbuf_a, sem_ga)
        issue(1, gbuf_b, sem_gb)
        issue(2, gbuf_c, sem_gc)

        def tri_body(p, carry2):
            u0 = p * 3
            for k in range(3):
                slot(u0 + k, k)
            return carry2

        lax.fori_loop(0, NPC // 3, tri_body, 0)
        slot(NPC - 2, (NPC - 2) % 3)
        slot(NPC - 1, (NPC - 1) % 3)
        wait_s(sbuf, sem_sa)
        return carry

    lax.fori_loop(0, NP, c_part, 0)
    plsc.subcore_barrier()

    def d_read(k, buf):
        st = pl.multiple_of(k * 16, 16)
        pltpu.sync_copy(acc_spm.at[pl.ds(st, 16)], buf)
        for e in range(16):
            for q in range(H // 16):
                sl = pl.ds(q * 16, 16)
                buf[e, sl] = jnp.maximum(buf[e, sl] + b_vm[sl], 0.0)

    def d_write(k, buf, sem):
        st = pl.multiple_of(k * 16, 16)
        pltpu.async_copy(buf, out.at[pl.ds(st, 16), pl.ds(c_idx * H, H)], sem)

    def d_wait(k, buf, sem):
        st = pl.multiple_of(k * 16, 16)
        pltpu.make_async_copy(buf, out.at[pl.ds(st, 16), pl.ds(c_idx * H, H)],
                              sem).wait()

    def d_pair(q, carry):
        ka = k0 + q * 2

        @pl.when(q > 0)
        def _():
            d_wait(ka - 2, gbuf_a, sem_ga)

        d_read(ka, gbuf_a)
        d_write(ka, gbuf_a, sem_ga)

        @pl.when(q > 0)
        def _():
            d_wait(ka - 1, gbuf_b, sem_gb)

        d_read(ka + 1, gbuf_b)
        d_write(ka + 1, gbuf_b, sem_gb)
        return carry

    lax.fori_loop(0, RPT // 2, d_pair, 0)
    d_wait(k0 + RPT - 3, gbuf_a, sem_ga)
    d_read(k0 + RPT - 1, gbuf_a)
    d_write(k0 + RPT - 1, gbuf_a, sem_ga)

    @pl.when(s_idx == NT - 1)
    def _():
        d_wait(k0 + RPT - 2, gbuf_b, sem_gb)
        d_read(NRC - 1, gbuf_b)
        d_write(NRC - 1, gbuf_b, sem_gb)

    d_wait(k0 + RPT - 1, gbuf_a, sem_ga)

    @pl.when(s_idx == NT - 1)
    def _():
        d_wait(NRC - 1, gbuf_b, sem_gb)

    @pl.when(s_idx != NT - 1)
    def _():
        d_wait(k0 + RPT - 2, gbuf_b, sem_gb)


def kernel(x, edge_index, edge_weight, W, b):
    x = x.astype(jnp.float32)
    W = W.astype(jnp.float32)
    ew = edge_weight.astype(jnp.float32)
    b = b.astype(jnp.float32)
    row = edge_index[0].astype(jnp.int32)
    col = edge_index[1].astype(jnp.int32)

    xlin = _xlin_split(x, W)
    row4 = row.reshape(NT, NP, NPC, CH)
    col4 = col.reshape(NT, NP, NPC, CH)
    ew4 = ew.reshape(NT, NP, NPC, CH)
    zeros_n = jnp.zeros((N,), jnp.float32)

    return _gcn_sc(xlin, row4, col4, ew4, zeros_n, b)


# speedup vs baseline: 2.0022x; 1.0805x over previous
import functools

import jax
import jax.numpy as jnp
from jax import lax
from jax.experimental import pallas as pl
from jax.experimental.pallas import tpu as pltpu
from jax.experimental.pallas import tpu_sc as plsc

N = 10000
E = 160000
D_IN = 256
D_OUT = 256
H = 128
NT = 16
EPT = E // NT
CH = 16
NP = 5
NPC = EPT // NP // CH
NRC = N // 16
RPT = NRC // NT


def _rsqrt16(d):
    i = lax.bitcast_convert_type(d, jnp.int32)
    i = jnp.int32(0x5F3759DF) - lax.shift_right_logical(i, 1)
    y = lax.bitcast_convert_type(i, jnp.float32)
    for _ in range(3):
        y = y * (1.5 - 0.5 * d * y * y)
    return y


def _mm_body(x_ref, w_ref, o_ref):
    o_ref[0] = jnp.dot(x_ref[...], w_ref[...], preferred_element_type=jnp.float32)


def _xlin_split(x, W):
    BN = 400
    out = pl.pallas_call(
        _mm_body,
        grid=(2, N // BN),
        in_specs=[
            pl.BlockSpec((BN, D_IN), lambda h, i: (i, 0)),
            pl.BlockSpec((D_IN, H), lambda h, i: (0, h)),
        ],
        out_specs=pl.BlockSpec((1, BN, H), lambda h, i: (h, i, 0)),
        out_shape=jax.ShapeDtypeStruct((2, N, H), jnp.float32),
    )(x, W)
    return out.reshape(2 * N, H)


_mesh = plsc.VectorSubcoreMesh(core_axis_name="c", subcore_axis_name="s")


@functools.partial(
    pl.kernel,
    out_type=jax.ShapeDtypeStruct((N, D_OUT), jnp.float32),
    mesh=_mesh,
    compiler_params=pltpu.CompilerParams(
        needs_layout_passes=False,
        use_tc_tiling_on_sc=False,
    ),
    scratch_types=[
        pltpu.VMEM_SHARED((N, H), jnp.float32),
        pltpu.VMEM_SHARED((N,), jnp.float32),
        pltpu.VMEM((N,), jnp.float32),
        pltpu.VMEM((NPC, CH), jnp.int32),
        pltpu.VMEM((NPC, CH), jnp.int32),
        pltpu.VMEM((NPC, CH), jnp.float32),
        pltpu.VMEM((CH, H), jnp.float32),
        pltpu.VMEM((CH, H), jnp.float32),
        pltpu.VMEM((CH, H), jnp.float32),
        pltpu.VMEM((CH, H), jnp.float32),
        pltpu.VMEM((CH, H), jnp.float32),
        pltpu.VMEM((16,), jnp.float32),
        pltpu.VMEM((H,), jnp.float32),
        pltpu.SemaphoreType.DMA,
        pltpu.SemaphoreType.DMA,
        pltpu.SemaphoreType.DMA,
        pltpu.SemaphoreType.DMA,
        pltpu.SemaphoreType.DMA,
    ],
)
def _gcn_sc(xlin, row4, col4, ew4, zeros_n, bvec, out,
            acc_spm, dg_spm, dis_v, row_a, col_a, ew_a,
            gbuf_a, gbuf_b, gbuf_c, gbuf_d, sbuf, dtmp, b_vm,
            sem_ga, sem_gb, sem_gc, sem_gd, sem_sa):
    c_idx = lax.axis_index("c")
    s_idx = lax.axis_index("s")
    half_base = c_idx * N

    pltpu.sync_copy(bvec.at[pl.ds(c_idx * H, H)], b_vm)

    @pl.when(s_idx == 0)
    def _():
        pltpu.sync_copy(zeros_n, dg_spm)

    plsc.subcore_barrier()

    def deg_part(hh, carry):
        pltpu.sync_copy(col4.at[s_idx, hh], col_a)
        pltpu.sync_copy(ew4.at[s_idx, hh], ew_a)

        def deg_group(g, carry2):
            base = g * 8
            for i in range(8):
                pltpu.async_copy(ew_a.at[base + i], dg_spm.at[col_a.at[base + i]],
                                 sem_ga, add=True)
            for i in range(8):
                pltpu.make_async_copy(ew_a.at[base + i],
                                      dg_spm.at[col_a.at[base + i]], sem_ga).wait()
            return carry2

        lax.fori_loop(0, NPC // 8, deg_group, 0)
        for u in range(NPC - NPC // 8 * 8):
            pltpu.sync_copy(ew_a.at[NPC // 8 * 8 + u],
                            dg_spm.at[col_a.at[NPC // 8 * 8 + u]], add=True)
        return carry

    lax.fori_loop(0, NP, deg_part, 0)
    plsc.subcore_barrier()

    def dis_chunk(k, carry):
        st = pl.multiple_of(k * 16, 16)
        pltpu.sync_copy(dg_spm.at[pl.ds(st, 16)], dtmp)
        dtmp[...] = _rsqrt16(dtmp[...] + 1.0)
        pltpu.sync_copy(dtmp, dg_spm.at[pl.ds(st, 16)])
        return carry

    lax.fori_loop(s_idx * RPT, (s_idx + 1) * RPT, dis_chunk, 0)

    @pl.when(s_idx == NT - 1)
    def _():
        dis_chunk(NRC - 1, 0)

    plsc.subcore_barrier()

    pltpu.sync_copy(dg_spm, dis_v)

    def b2_issue(k, buf, sem):
        st = pl.multiple_of(k * 16, 16)
        pltpu.async_copy(xlin.at[pl.ds(half_base + st, 16)], buf, sem)

    def b2_wait(k, buf, sem):
        st = pl.multiple_of(k * 16, 16)
        pltpu.make_async_copy(xlin.at[pl.ds(half_base + st, 16)], buf, sem).wait()

    def b2_process(k, buf):
        st = pl.multiple_of(k * 16, 16)
        dv = dis_v[pl.ds(st, 16)]
        for e in range(16):
            dse = dv[e]
            s2 = dse * dse
            for q in range(H // 16):
                sl = pl.ds(q * 16, 16)
                buf[e, sl] = buf[e, sl] * s2
        pltpu.sync_copy(buf, acc_spm.at[pl.ds(st, 16)])

    k0 = s_idx * RPT
    b2_issue(k0, gbuf_a, sem_ga)

    def b2_pair(q, carry):
        ka = k0 + q * 2
        b2_wait(ka, gbuf_a, sem_ga)
        b2_issue(ka + 1, gbuf_b, sem_gb)
        b2_process(ka, gbuf_a)
        b2_wait(ka + 1, gbuf_b, sem_gb)
        b2_issue(ka + 2, gbuf_a, sem_ga)
        b2_process(ka + 1, gbuf_b)
        return carry

    lax.fori_loop(0, RPT // 2, b2_pair, 0)
    b2_wait(k0 + RPT - 1, gbuf_a, sem_ga)
    b2_process(k0 + RPT - 1, gbuf_a)

    @pl.when(s_idx == NT - 1)
    def _():
        st = pl.multiple_of((NRC - 1) * 16, 16)
        pltpu.sync_copy(xlin.at[pl.ds(half_base + st, 16)], gbuf_a)
        b2_process(NRC - 1, gbuf_a)

    plsc.subcore_barrier()

    def issue(u, buf, sem):
        pltpu.async_copy(xlin.at[row_a[u]], buf, sem)

    def wait_g(u, buf, sem):
        pltpu.make_async_copy(xlin.at[row_a[u]], buf, sem).wait()

    def scat(u, buf, sem):
        pltpu.async_copy(buf, acc_spm.at[col_a.at[u]], sem, add=True)

    def wait_s(buf, sem):
        pltpu.make_async_copy(buf, acc_spm.at[col_a.at[0]], sem).wait()

    gbufs = (gbuf_a, gbuf_b, gbuf_c, gbuf_d)
    gsems = (sem_ga, sem_gb, sem_gc, sem_gd)

    def slot(u, gi):
        gb, gs = gbufs[gi], gsems[gi]
        wait_g(u, gb, gs)

        @pl.when(u >= 1)
        def _():
            wait_s(sbuf, sem_sa)

        norm16 = ew_a[u]
        for e in range(CH):
            ne = norm16[e]
            for q in range(H // 16):
                sl = pl.ds(q * 16, 16)
                sbuf[e, sl] = gb[e, sl] * ne

        @pl.when(u + 4 < NPC)
        def _():
            issue(u + 4, gb, gs)

        scat(u, sbuf, sem_sa)

    def c_part(hh, carry):
        pltpu.sync_copy(row4.at[s_idx, hh], row_a)
        pltpu.sync_copy(col4.at[s_idx, hh], col_a)
        pltpu.sync_copy(ew4.at[s_idx, hh], ew_a)

        def prep_chunk(u, carry2):
            r16 = row_a[u]
            c16 = col_a[u]
            w16 = ew_a[u]
            dr = plsc.load_gather(dis_v, [r16])
            dc = plsc.load_gather(dis_v, [c16])
            ew_a[u] = dr * w16 * dc
            row_a[u] = r16 + half_base
            return carry2

        lax.fori_loop(0, NPC, prep_chunk, 0)

        issue(0, gbuf_a, sem_ga)
        issue(1, gbuf_b, sem_gb)
        issue(2, gbuf_c, sem_gc)
        issue(3, gbuf_d, sem_gd)

        def quad_body(p, carry2):
            u0 = p * 4
            for k in range(4):
                slot(u0 + k, k)
            return carry2

        lax.fori_loop(0, NPC // 4, quad_body, 0)
        slot(NPC - 1, (NPC - 1) % 4)
        wait_s(sbuf, sem_sa)
        return carry

    lax.fori_loop(0, NP, c_part, 0)
    plsc.subcore_barrier()

    def d_read(k, buf):
        st = pl.multiple_of(k * 16, 16)
        pltpu.sync_copy(acc_spm.at[pl.ds(st, 16)], buf)
        for e in range(16):
            for q in range(H // 16):
                sl = pl.ds(q * 16, 16)
                buf[e, sl] = jnp.maximum(buf[e, sl] + b_vm[sl], 0.0)

    def d_write(k, buf, sem):
        st = pl.multiple_of(k * 16, 16)
        pltpu.async_copy(buf, out.at[pl.ds(st, 16), pl.ds(c_idx * H, H)], sem)

    def d_wait(k, buf, sem):
        st = pl.multiple_of(k * 16, 16)
        pltpu.make_async_copy(buf, out.at[pl.ds(st, 16), pl.ds(c_idx * H, H)],
                              sem).wait()

    def d_pair(q, carry):
        ka = k0 + q * 2

        @pl.when(q > 0)
        def _():
            d_wait(ka - 2, gbuf_a, sem_ga)

        d_read(ka, gbuf_a)
        d_write(ka, gbuf_a, sem_ga)

        @pl.when(q > 0)
        def _():
            d_wait(ka - 1, gbuf_b, sem_gb)

        d_read(ka + 1, gbuf_b)
        d_write(ka + 1, gbuf_b, sem_gb)
        return carry

    lax.fori_loop(0, RPT // 2, d_pair, 0)
    d_wait(k0 + RPT - 3, gbuf_a, sem_ga)
    d_read(k0 + RPT - 1, gbuf_a)
    d_write(k0 + RPT - 1, gbuf_a, sem_ga)

    @pl.when(s_idx == NT - 1)
    def _():
        d_wait(k0 + RPT - 2, gbuf_b, sem_gb)
        d_read(NRC - 1, gbuf_b)
        d_write(NRC - 1, gbuf_b, sem_gb)

    d_wait(k0 + RPT - 1, gbuf_a, sem_ga)

    @pl.when(s_idx == NT - 1)
    def _():
        d_wait(NRC - 1, gbuf_b, sem_gb)

    @pl.when(s_idx != NT - 1)
    def _():
        d_wait(k0 + RPT - 2, gbuf_b, sem_gb)


def kernel(x, edge_index, edge_weight, W, b):
    x = x.astype(jnp.float32)
    W = W.astype(jnp.float32)
    ew = edge_weight.astype(jnp.float32)
    b = b.astype(jnp.float32)
    row = edge_index[0].astype(jnp.int32)
    col = edge_index[1].astype(jnp.int32)

    xlin = _xlin_split(x, W)
    row4 = row.reshape(NT, NP, NPC, CH)
    col4 = col.reshape(NT, NP, NPC, CH)
    ew4 = ew.reshape(NT, NP, NPC, CH)
    zeros_n = jnp.zeros((N,), jnp.float32)

    return _gcn_sc(xlin, row4, col4, ew4, zeros_n, b)


# speedup vs baseline: 2.1027x; 1.0502x over previous
import functools

import jax
import jax.numpy as jnp
from jax import lax
from jax.experimental import pallas as pl
from jax.experimental.pallas import tpu as pltpu
from jax.experimental.pallas import tpu_sc as plsc

N = 10000
E = 160000
D_IN = 256
D_OUT = 256
H = 128
NT = 16
EPT = E // NT
CH = 16
NP = 5
NPC = EPT // NP // CH
NRC = N // 16
RPT = NRC // NT


def _rsqrt16(d):
    i = lax.bitcast_convert_type(d, jnp.int32)
    i = jnp.int32(0x5F3759DF) - lax.shift_right_logical(i, 1)
    y = lax.bitcast_convert_type(i, jnp.float32)
    for _ in range(3):
        y = y * (1.5 - 0.5 * d * y * y)
    return y


def _mm_body(x_ref, w_ref, o_ref):
    o_ref[0] = jnp.dot(x_ref[...], w_ref[...], preferred_element_type=jnp.float32)


def _xlin_split(x, W):
    BN = 400
    out = pl.pallas_call(
        _mm_body,
        grid=(2, N // BN),
        in_specs=[
            pl.BlockSpec((BN, D_IN), lambda h, i: (i, 0)),
            pl.BlockSpec((D_IN, H), lambda h, i: (0, h)),
        ],
        out_specs=pl.BlockSpec((1, BN, H), lambda h, i: (h, i, 0)),
        out_shape=jax.ShapeDtypeStruct((2, N, H), jnp.float32),
    )(x, W)
    return out.reshape(2 * N, H)


_mesh = plsc.VectorSubcoreMesh(core_axis_name="c", subcore_axis_name="s")


@functools.partial(
    pl.kernel,
    out_type=jax.ShapeDtypeStruct((N, D_OUT), jnp.float32),
    mesh=_mesh,
    compiler_params=pltpu.CompilerParams(
        needs_layout_passes=False,
        use_tc_tiling_on_sc=False,
    ),
    scratch_types=[
        pltpu.VMEM_SHARED((N, H), jnp.float32),
        pltpu.VMEM_SHARED((N,), jnp.float32),
        pltpu.VMEM((N,), jnp.float32),
        pltpu.VMEM((NPC, CH), jnp.int32),
        pltpu.VMEM((NPC, CH), jnp.int32),
        pltpu.VMEM((NPC, CH), jnp.float32),
        pltpu.VMEM((CH, H), jnp.float32),
        pltpu.VMEM((CH, H), jnp.float32),
        pltpu.VMEM((CH, H), jnp.float32),
        pltpu.VMEM((CH, H), jnp.float32),
        pltpu.VMEM((CH, H), jnp.float32),
        pltpu.VMEM((CH, H), jnp.float32),
        pltpu.VMEM((16,), jnp.float32),
        pltpu.VMEM((H,), jnp.float32),
        pltpu.SemaphoreType.DMA,
        pltpu.SemaphoreType.DMA,
        pltpu.SemaphoreType.DMA,
        pltpu.SemaphoreType.DMA,
        pltpu.SemaphoreType.DMA,
        pltpu.SemaphoreType.DMA,
    ],
)
def _gcn_sc(xlin, row4, col4, ew4, zeros_n, bvec, out,
            acc_spm, dg_spm, dis_v, row_a, col_a, ew_a,
            gbuf_a, gbuf_b, gbuf_c, gbuf_d, sbuf, sbuf2, dtmp, b_vm,
            sem_ga, sem_gb, sem_gc, sem_gd, sem_sa, sem_sb):
    c_idx = lax.axis_index("c")
    s_idx = lax.axis_index("s")
    half_base = c_idx * N

    pltpu.sync_copy(bvec.at[pl.ds(c_idx * H, H)], b_vm)

    @pl.when(s_idx == 0)
    def _():
        pltpu.sync_copy(zeros_n, dg_spm)

    plsc.subcore_barrier()

    def deg_part(hh, carry):
        pltpu.sync_copy(col4.at[s_idx, hh], col_a)
        pltpu.sync_copy(ew4.at[s_idx, hh], ew_a)

        def deg_group(g, carry2):
            base = g * 8
            for i in range(8):
                pltpu.async_copy(ew_a.at[base + i], dg_spm.at[col_a.at[base + i]],
                                 sem_ga, add=True)
            for i in range(8):
                pltpu.make_async_copy(ew_a.at[base + i],
                                      dg_spm.at[col_a.at[base + i]], sem_ga).wait()
            return carry2

        lax.fori_loop(0, NPC // 8, deg_group, 0)
        for u in range(NPC - NPC // 8 * 8):
            pltpu.sync_copy(ew_a.at[NPC // 8 * 8 + u],
                            dg_spm.at[col_a.at[NPC // 8 * 8 + u]], add=True)
        return carry

    lax.fori_loop(0, NP, deg_part, 0)
    plsc.subcore_barrier()

    def dis_chunk(k, carry):
        st = pl.multiple_of(k * 16, 16)
        pltpu.sync_copy(dg_spm.at[pl.ds(st, 16)], dtmp)
        dtmp[...] = _rsqrt16(dtmp[...] + 1.0)
        pltpu.sync_copy(dtmp, dg_spm.at[pl.ds(st, 16)])
        return carry

    lax.fori_loop(s_idx * RPT, (s_idx + 1) * RPT, dis_chunk, 0)

    @pl.when(s_idx == NT - 1)
    def _():
        dis_chunk(NRC - 1, 0)

    plsc.subcore_barrier()

    pltpu.sync_copy(dg_spm, dis_v)

    def b2_issue(k, buf, sem):
        st = pl.multiple_of(k * 16, 16)
        pltpu.async_copy(xlin.at[pl.ds(half_base + st, 16)], buf, sem)

    def b2_wait(k, buf, sem):
        st = pl.multiple_of(k * 16, 16)
        pltpu.make_async_copy(xlin.at[pl.ds(half_base + st, 16)], buf, sem).wait()

    def b2_process(k, buf):
        st = pl.multiple_of(k * 16, 16)
        dv = dis_v[pl.ds(st, 16)]
        for e in range(16):
            dse = dv[e]
            s2 = dse * dse
            for q in range(H // 16):
                sl = pl.ds(q * 16, 16)
                buf[e, sl] = buf[e, sl] * s2
        pltpu.sync_copy(buf, acc_spm.at[pl.ds(st, 16)])

    k0 = s_idx * RPT
    b2_issue(k0, gbuf_a, sem_ga)

    def b2_pair(q, carry):
        ka = k0 + q * 2
        b2_wait(ka, gbuf_a, sem_ga)
        b2_issue(ka + 1, gbuf_b, sem_gb)
        b2_process(ka, gbuf_a)
        b2_wait(ka + 1, gbuf_b, sem_gb)
        b2_issue(ka + 2, gbuf_a, sem_ga)
        b2_process(ka + 1, gbuf_b)
        return carry

    lax.fori_loop(0, RPT // 2, b2_pair, 0)
    b2_wait(k0 + RPT - 1, gbuf_a, sem_ga)
    b2_process(k0 + RPT - 1, gbuf_a)

    @pl.when(s_idx == NT - 1)
    def _():
        st = pl.multiple_of((NRC - 1) * 16, 16)
        pltpu.sync_copy(xlin.at[pl.ds(half_base + st, 16)], gbuf_a)
        b2_process(NRC - 1, gbuf_a)

    plsc.subcore_barrier()

    def issue(u, buf, sem):
        pltpu.async_copy(xlin.at[row_a[u]], buf, sem)

    def wait_g(u, buf, sem):
        pltpu.make_async_copy(xlin.at[row_a[u]], buf, sem).wait()

    def scat(u, buf, sem):
        pltpu.async_copy(buf, acc_spm.at[col_a.at[u]], sem, add=True)

    def wait_s(buf, sem):
        pltpu.make_async_copy(buf, acc_spm.at[col_a.at[0]], sem).wait()

    gbufs = (gbuf_a, gbuf_b, gbuf_c, gbuf_d)
    gsems = (sem_ga, sem_gb, sem_gc, sem_gd)
    sbufs = (sbuf, sbuf2)
    ssems = (sem_sa, sem_sb)

    def slot(u, gi, sj):
        gb, gs = gbufs[gi], gsems[gi]
        sb, ss = sbufs[sj], ssems[sj]
        wait_g(u, gb, gs)

        @pl.when(u >= 2)
        def _():
            wait_s(sb, ss)

        norm16 = ew_a[u]
        for e in range(CH):
            ne = norm16[e]
            for q in range(H // 16):
                sl = pl.ds(q * 16, 16)
                sb[e, sl] = gb[e, sl] * ne

        @pl.when(u + 4 < NPC)
        def _():
            issue(u + 4, gb, gs)

        scat(u, sb, ss)

    def c_part(hh, carry):
        pltpu.sync_copy(row4.at[s_idx, hh], row_a)
        pltpu.sync_copy(col4.at[s_idx, hh], col_a)
        pltpu.sync_copy(ew4.at[s_idx, hh], ew_a)

        def prep_chunk(u, carry2):
            r16 = row_a[u]
            c16 = col_a[u]
            w16 = ew_a[u]
            dr = plsc.load_gather(dis_v, [r16])
            dc = plsc.load_gather(dis_v, [c16])
            ew_a[u] = dr * w16 * dc
            row_a[u] = r16 + half_base
            return carry2

        lax.fori_loop(0, NPC, prep_chunk, 0)

        issue(0, gbuf_a, sem_ga)
        issue(1, gbuf_b, sem_gb)
        issue(2, gbuf_c, sem_gc)
        issue(3, gbuf_d, sem_gd)

        def quad_body(p, carry2):
            u0 = p * 4
            for k in range(4):
                slot(u0 + k, k, k % 2)
            return carry2

        lax.fori_loop(0, NPC // 4, quad_body, 0)
        slot(NPC - 1, (NPC - 1) % 4, (NPC - 1) % 2)
        wait_s(sbuf2, sem_sb)
        wait_s(sbuf, sem_sa)
        return carry

    lax.fori_loop(0, NP, c_part, 0)
    plsc.subcore_barrier()

    def d_read(k, buf):
        st = pl.multiple_of(k * 16, 16)
        pltpu.sync_copy(acc_spm.at[pl.ds(st, 16)], buf)
        for e in range(16):
            for q in range(H // 16):
                sl = pl.ds(q * 16, 16)
                buf[e, sl] = jnp.maximum(buf[e, sl] + b_vm[sl], 0.0)

    def d_write(k, buf, sem):
        st = pl.multiple_of(k * 16, 16)
        pltpu.async_copy(buf, out.at[pl.ds(st, 16), pl.ds(c_idx * H, H)], sem)

    def d_wait(k, buf, sem):
        st = pl.multiple_of(k * 16, 16)
        pltpu.make_async_copy(buf, out.at[pl.ds(st, 16), pl.ds(c_idx * H, H)],
                              sem).wait()

    def d_pair(q, carry):
        ka = k0 + q * 2

        @pl.when(q > 0)
        def _():
            d_wait(ka - 2, gbuf_a, sem_ga)

        d_read(ka, gbuf_a)
        d_write(ka, gbuf_a, sem_ga)

        @pl.when(q > 0)
        def _():
            d_wait(ka - 1, gbuf_b, sem_gb)

        d_read(ka + 1, gbuf_b)
        d_write(ka + 1, gbuf_b, sem_gb)
        return carry

    lax.fori_loop(0, RPT // 2, d_pair, 0)
    d_wait(k0 + RPT - 3, gbuf_a, sem_ga)
    d_read(k0 + RPT - 1, gbuf_a)
    d_write(k0 + RPT - 1, gbuf_a, sem_ga)

    @pl.when(s_idx == NT - 1)
    def _():
        d_wait(k0 + RPT - 2, gbuf_b, sem_gb)
        d_read(NRC - 1, gbuf_b)
        d_write(NRC - 1, gbuf_b, sem_gb)

    d_wait(k0 + RPT - 1, gbuf_a, sem_ga)

    @pl.when(s_idx == NT - 1)
    def _():
        d_wait(NRC - 1, gbuf_b, sem_gb)

    @pl.when(s_idx != NT - 1)
    def _():
        d_wait(k0 + RPT - 2, gbuf_b, sem_gb)


def kernel(x, edge_index, edge_weight, W, b):
    x = x.astype(jnp.float32)
    W = W.astype(jnp.float32)
    ew = edge_weight.astype(jnp.float32)
    b = b.astype(jnp.float32)
    row = edge_index[0].astype(jnp.int32)
    col = edge_index[1].astype(jnp.int32)

    xlin = _xlin_split(x, W)
    row4 = row.reshape(NT, NP, NPC, CH)
    col4 = col.reshape(NT, NP, NPC, CH)
    ew4 = ew.reshape(NT, NP, NPC, CH)
    zeros_n = jnp.zeros((N,), jnp.float32)

    return _gcn_sc(xlin, row4, col4, ew4, zeros_n, b)
